# baseline (speedup 1.0000x reference)
import functools
import jax
import jax.numpy as jnp
from jax import lax
from jax.experimental import pallas as pl
from jax.experimental.pallas import tpu as pltpu
from jax.experimental.pallas import tpu_sc as plsc

D = 768
SEQ = 20
L = 16
NV = D // L
DB = D // 128
NC = 2
NS = 16
NW = NC * NS
C = 40
EPS = 1e-3


def _rsqrt(x):
    i = lax.bitcast_convert_type(x, jnp.int32)
    i = jnp.int32(0x5F3759DF) - lax.shift_right_logical(i, 1)
    y = lax.bitcast_convert_type(i, jnp.float32)
    for _ in range(3):
        y = y * (1.5 - 0.5 * x * y * y)
    return y


def _make_kernel(B, S):
    N = B * S
    per_w = N // NW
    n_chunks = per_w // C
    rows_per_chunk = C // S
    row5 = (1, DB, 1, 128)
    mesh = plsc.VectorSubcoreMesh(
        core_axis_name="c", subcore_axis_name="s",
        num_cores=NC, num_subcores=NS)

    @functools.partial(
        pl.kernel,
        out_type=jax.ShapeDtypeStruct((S, B // 8, DB, 8, 128), jnp.float32),
        mesh=mesh,
        compiler_params=pltpu.CompilerParams(
            use_tc_tiling_on_sc=False, needs_layout_passes=False),
        scratch_types=[
            pltpu.VMEM((per_w,), jnp.int32),
            pltpu.VMEM((per_w + L,), jnp.int32),
            pltpu.VMEM((2 * SEQ, D), jnp.float32),
            pltpu.VMEM((C,) + row5, jnp.float32),
            pltpu.VMEM((C,) + row5, jnp.float32),
            pltpu.VMEM((D,), jnp.float32),
            pltpu.VMEM((D,), jnp.float32),
            pltpu.SMEM((C,), jnp.float32),
            pltpu.SMEM((C,), jnp.float32),
            pltpu.SemaphoreType.DMA,
            pltpu.SemaphoreType.DMA,
            pltpu.SemaphoreType.DMA,
            pltpu.SemaphoreType.DMA,
        ],
    )
    def k(x_hbm, kk_hbm, tok_hbm, pos_hbm, segt_hbm, g_hbm, bt_hbm, out_hbm,
          idx_v, kk_v, combo, buf0, buf1, g_v, b_v, a_s, b_s,
          gs0, gs1, os0, os1):
        wid = lax.axis_index("s") * NC + lax.axis_index("c")
        wbase = wid * per_w

        pltpu.sync_copy(x_hbm.at[pl.ds(wbase, per_w)], idx_v)
        pltpu.sync_copy(kk_hbm.at[pl.ds(wbase, per_w)],
                        kk_v.at[pl.ds(0, per_w)])
        pltpu.sync_copy(g_hbm, g_v)
        pltpu.sync_copy(bt_hbm, b_v)
        pltpu.sync_copy(pos_hbm.at[pl.ds(0, SEQ)], buf0.at[pl.ds(0, SEQ)])
        pltpu.sync_copy(segt_hbm, buf0.at[pl.ds(SEQ, 2)])

        @plsc.parallel_loop(0, SEQ)
        def build_combo(p):
            for s2 in range(2):
                for j in range(NV):
                    jb, jo = j // 8, (j % 8) * L
                    sl = pl.ds(jo, L)
                    combo[s2 * SEQ + p, pl.ds(L * j, L)] = (
                        buf0[p, 0, jb, 0, sl] + buf0[SEQ + s2, 0, jb, 0, sl])

        def gather(g, buf, gsem):
            idx = idx_v.at[pl.ds(g * C, C)]
            pltpu.async_copy(tok_hbm.at[idx], buf, gsem)

        def wait_gather(buf, gsem):
            pltpu.make_async_copy(tok_hbm.at[idx_v.at[pl.ds(0, C)]],
                                  buf, gsem).wait()

        def out_slices(g, buf):
            b0 = (wbase + g * C) // S
            res = []
            for k2 in range(rows_per_chunk):
                b = b0 + k2
                src = buf.at[pl.ds(k2 * S, S)]
                dst = out_hbm.at[pl.ds(0, S), pl.ds(b // 8, 1), :,
                                 pl.ds(b % 8, 1), :]
                res.append((src, dst))
            return res

        def drain_out(g, buf, osem):
            for src, dst in out_slices(g, buf):
                pltpu.make_async_copy(src, dst, osem).wait()

        def compute_and_store(g, buf, osem):
            @plsc.parallel_loop(0, C, unroll=2)
            def pass1(t):
                kk = kk_v[pl.ds(g * C + t, L)][0]
                acc = jnp.zeros((L,), jnp.float32)
                acc2 = jnp.zeros((L,), jnp.float32)
                for j in range(NV):
                    jb, jo = j // 8, (j % 8) * L
                    sl = pl.ds(jo, L)
                    v = buf[t, 0, jb, 0, sl] + combo[kk, pl.ds(L * j, L)]
                    buf[t, 0, jb, 0, sl] = v
                    acc = acc + v
                    acc2 = acc2 + v * v
                s1 = jnp.sum(acc)
                s2 = jnp.sum(acc2)
                mean = s1 * (1.0 / D)
                var = s2 * (1.0 / D) - mean * mean
                inv = _rsqrt(var + EPS)
                a_s[t] = inv
                b_s[t] = -mean * inv

            @plsc.parallel_loop(0, NV, unroll=2)
            def pass2(j):
                jb = j // 8
                jo = (j % 8) * L
                gv = g_v[pl.ds(L * j, L)]
                bv = b_v[pl.ds(L * j, L)]
                @plsc.parallel_loop(0, C // 8, unroll=2)
                def rows(tb):
                    for u in range(8):
                        t = tb * 8 + u
                        val = buf[t, 0, jb, 0, pl.ds(jo, L)]
                        buf[t, 0, jb, 0, pl.ds(jo, L)] = (
                            (val * a_s[t] + b_s[t]) * gv + bv)

            for src, dst in out_slices(g, buf):
                pltpu.async_copy(src, dst, osem)

        gather(0, buf0, gs0)

        def chunk_pair(gp, carry):
            g0 = gp * 2
            g1 = g0 + 1

            @pl.when(g0 >= 1)
            def _():
                drain_out(g0 - 1, buf1, os1)
            gather(g1, buf1, gs1)
            wait_gather(buf0, gs0)
            compute_and_store(g0, buf0, os0)

            @pl.when(g1 + 1 < n_chunks)
            def _():
                drain_out(g0, buf0, os0)
                gather(g1 + 1, buf0, gs0)
            wait_gather(buf1, gs1)
            compute_and_store(g1, buf1, os1)
            return carry

        lax.fori_loop(0, n_chunks // 2, chunk_pair, None)
        drain_out(0, buf0, os0)
        drain_out(0, buf1, os1)

    return k


def kernel(x, seg, tok_table, pos_table, seg_table, gamma, beta):
    B, S = x.shape
    N = B * S
    kk = (seg * SEQ + jnp.arange(S, dtype=jnp.int32)[None, :]).reshape(N)
    k = _make_kernel(B, S)
    out5 = k(x.reshape(N), kk,
             tok_table.reshape(tok_table.shape[0], 1, DB, 1, 128),
             pos_table.reshape(pos_table.shape[0], 1, DB, 1, 128),
             seg_table.reshape(seg_table.shape[0], 1, DB, 1, 128),
             gamma, beta)
    return out5.transpose(1, 3, 0, 2, 4).reshape(B, S, D)


# speedup vs baseline: 1.8505x; 1.8505x over previous
import functools
import jax
import jax.numpy as jnp
from jax import lax
from jax.experimental import pallas as pl
from jax.experimental.pallas import tpu as pltpu
from jax.experimental.pallas import tpu_sc as plsc

D = 768
SEQ = 20
L = 16
NV = D // L
DB = D // 128
NC = 2
NS = 16
NW = NC * NS
C = 40
EPS = 1e-3


def _rsqrt(x):
    i = lax.bitcast_convert_type(x, jnp.int32)
    i = jnp.int32(0x5F3759DF) - lax.shift_right_logical(i, 1)
    y = lax.bitcast_convert_type(i, jnp.float32)
    for _ in range(3):
        y = y * (1.5 - 0.5 * x * y * y)
    return y


def _make_kernel(B, S):
    N = B * S
    per_w = N // NW
    n_chunks = per_w // C
    rows_per_chunk = C // S
    row5 = (1, DB, 1, 128)
    mesh = plsc.VectorSubcoreMesh(
        core_axis_name="c", subcore_axis_name="s",
        num_cores=NC, num_subcores=NS)

    @functools.partial(
        pl.kernel,
        out_type=jax.ShapeDtypeStruct((S, B // 8, DB, 8, 128), jnp.float32),
        mesh=mesh,
        compiler_params=pltpu.CompilerParams(
            use_tc_tiling_on_sc=False, needs_layout_passes=False),
        scratch_types=[
            pltpu.VMEM((per_w,), jnp.int32),
            pltpu.VMEM((per_w + L,), jnp.int32),
            pltpu.VMEM((2 * SEQ, D), jnp.float32),
            pltpu.VMEM((C,) + row5, jnp.float32),
            pltpu.VMEM((C,) + row5, jnp.float32),
            pltpu.VMEM((D,), jnp.float32),
            pltpu.VMEM((D,), jnp.float32),
            pltpu.SMEM((C,), jnp.float32),
            pltpu.SMEM((C,), jnp.float32),
            pltpu.SemaphoreType.DMA,
            pltpu.SemaphoreType.DMA,
            pltpu.SemaphoreType.DMA,
            pltpu.SemaphoreType.DMA,
        ],
    )
    def k(x_hbm, kk_hbm, tok_hbm, pos_hbm, segt_hbm, g_hbm, bt_hbm, out_hbm,
          idx_v, kk_v, combo, buf0, buf1, g_v, b_v, a_s, b_s,
          gs0, gs1, os0, os1):
        wid = lax.axis_index("s") * NC + lax.axis_index("c")
        wbase = wid * per_w

        pltpu.sync_copy(x_hbm.at[pl.ds(wbase, per_w)], idx_v)
        pltpu.sync_copy(kk_hbm.at[pl.ds(wbase, per_w)],
                        kk_v.at[pl.ds(0, per_w)])
        pltpu.sync_copy(g_hbm, g_v)
        pltpu.sync_copy(bt_hbm, b_v)
        pltpu.sync_copy(pos_hbm.at[pl.ds(0, SEQ)], buf0.at[pl.ds(0, SEQ)])
        pltpu.sync_copy(segt_hbm, buf0.at[pl.ds(SEQ, 2)])

        @plsc.parallel_loop(0, SEQ)
        def build_combo(p):
            for s2 in range(2):
                for j in range(NV):
                    jb, jo = j // 8, (j % 8) * L
                    sl = pl.ds(jo, L)
                    combo[s2 * SEQ + p, pl.ds(L * j, L)] = (
                        buf0[p, 0, jb, 0, sl] + buf0[SEQ + s2, 0, jb, 0, sl])

        def gather(g, buf, gsem):
            idx = idx_v.at[pl.ds(g * C, C)]
            pltpu.async_copy(tok_hbm.at[idx], buf, gsem)

        def wait_gather(buf, gsem):
            pltpu.make_async_copy(tok_hbm.at[idx_v.at[pl.ds(0, C)]],
                                  buf, gsem).wait()

        def out_slices(g, buf):
            b0 = (wbase + g * C) // S
            res = []
            for k2 in range(rows_per_chunk):
                b = b0 + k2
                src = buf.at[pl.ds(k2 * S, S)]
                dst = out_hbm.at[pl.ds(0, S), pl.ds(b // 8, 1), :,
                                 pl.ds(b % 8, 1), :]
                res.append((src, dst))
            return res

        def drain_out(g, buf, osem):
            for src, dst in out_slices(g, buf):
                pltpu.make_async_copy(src, dst, osem).wait()

        def compute_and_store(g, buf, osem):
            @plsc.parallel_loop(0, C, unroll=2)
            def pass1(t):
                kk = kk_v[pl.ds(g * C + t, L)][0]
                acc = jnp.zeros((L,), jnp.float32)
                acc2 = jnp.zeros((L,), jnp.float32)
                for j in range(NV):
                    jb, jo = j // 8, (j % 8) * L
                    sl = pl.ds(jo, L)
                    v = buf[t, 0, jb, 0, sl] + combo[kk, pl.ds(L * j, L)]
                    buf[t, 0, jb, 0, sl] = v
                    acc = acc + v
                    acc2 = acc2 + v * v
                s1 = jnp.sum(acc)
                s2 = jnp.sum(acc2)
                mean = s1 * (1.0 / D)
                var = s2 * (1.0 / D) - mean * mean
                inv = _rsqrt(var + EPS)
                a_s[t] = inv
                b_s[t] = -mean * inv

            @plsc.parallel_loop(0, NV, unroll=2)
            def pass2(j):
                jb = j // 8
                jo = (j % 8) * L
                gv = g_v[pl.ds(L * j, L)]
                bv = b_v[pl.ds(L * j, L)]
                @plsc.parallel_loop(0, C // 8)
                def rows(tb):
                    for u in range(8):
                        t = tb * 8 + u
                        val = buf[t, 0, jb, 0, pl.ds(jo, L)]
                        buf[t, 0, jb, 0, pl.ds(jo, L)] = (
                            (val * a_s[t] + b_s[t]) * gv + bv)

            for src, dst in out_slices(g, buf):
                pltpu.async_copy(src, dst, osem)

        gather(0, buf0, gs0)

        def chunk_pair(gp, carry):
            g0 = gp * 2
            g1 = g0 + 1

            @pl.when(g0 >= 1)
            def _():
                drain_out(g0 - 1, buf1, os1)
            gather(g1, buf1, gs1)
            wait_gather(buf0, gs0)
            compute_and_store(g0, buf0, os0)

            @pl.when(g1 + 1 < n_chunks)
            def _():
                drain_out(g0, buf0, os0)
                gather(g1 + 1, buf0, gs0)
            wait_gather(buf1, gs1)
            compute_and_store(g1, buf1, os1)
            return carry

        lax.fori_loop(0, n_chunks // 2, chunk_pair, None)
        drain_out(0, buf0, os0)
        drain_out(0, buf1, os1)

    return k


def kernel(x, seg, tok_table, pos_table, seg_table, gamma, beta):
    B, S = x.shape
    N = B * S
    kk = (seg * SEQ + jnp.arange(S, dtype=jnp.int32)[None, :]).reshape(N)
    k = _make_kernel(B, S)
    out5 = k(x.reshape(N), kk,
             tok_table.reshape(tok_table.shape[0], 1, DB, 1, 128),
             pos_table.reshape(pos_table.shape[0], 1, DB, 1, 128),
             seg_table.reshape(seg_table.shape[0], 1, DB, 1, 128),
             gamma, beta)
    return out5.transpose(1, 3, 0, 2, 4).reshape(B, S, D)
